# Initial kernel scaffold; baseline (speedup 1.0000x reference)
import jax
import jax.numpy as jnp
from jax.experimental import pallas as pl


def kernel(vis_features, lang_features, W1_v, W2_v, W1_l, W2_l):
    raise NotImplementedError("write your pallas kernel here")


# speedup vs baseline: 1.1799x; 1.1799x over previous
import functools

import jax
import jax.numpy as jnp
from jax.experimental import pallas as pl
from jax.experimental.pallas import tpu as pltpu

B = 4096
D_FEAT = 1024
D_BN = 256
D_SHARED = 256

_R1 = 512
_R2 = 512

_DN = (((1,), (1,)), ((), ()))


def _proj_rows(x, W1, W2):
    h = jax.lax.dot_general(x, W1, _DN, preferred_element_type=jnp.float32)
    h = jnp.maximum(h, 0.0)
    o = jax.lax.dot_general(h, W2, _DN, preferred_element_type=jnp.float32)
    n = jnp.sqrt(jnp.sum(o * o, axis=-1, keepdims=True))
    return o / jnp.maximum(n, 1e-12)


def _project_kernel(vis_ref, lang_ref, w1v_ref, w2v_ref, w1l_ref, w2l_ref,
                    vout_ref, lout_ref):
    vout_ref[...] = _proj_rows(vis_ref[...], w1v_ref[...], w2v_ref[...])
    lout_ref[...] = _proj_rows(lang_ref[...], w1l_ref[...], w2l_ref[...])


def _sim_kernel(vp_ref, lp_ref, out_ref):
    i = pl.program_id(0)

    @pl.when(i == 0)
    def _init():
        out_ref[0] = 0.0
        out_ref[1] = 0.0

    sim = jax.lax.dot_general(vp_ref[...], lp_ref[...], _DN,
                              preferred_element_type=jnp.float32)
    rows = i * _R2 + jax.lax.broadcasted_iota(jnp.int32, (_R2, B), 0)
    cols = jax.lax.broadcasted_iota(jnp.int32, (_R2, B), 1)
    eye = rows == cols
    pos = jnp.sum(jnp.where(eye, sim, 0.0), axis=1)
    neg = jnp.max(jnp.where(eye, -1e9, sim), axis=1)
    out_ref[0] = out_ref[0] + jnp.sum(pos) * (1.0 / B)
    out_ref[1] = out_ref[1] + jnp.sum(pos - neg) * (1.0 / B)


@jax.jit
def kernel(vis_features, lang_features, W1_v, W2_v, W1_l, W2_l):
    full = lambda shape: pl.BlockSpec(shape, lambda i: (0, 0))
    vis_proj, lang_proj = pl.pallas_call(
        _project_kernel,
        grid=(B // _R1,),
        in_specs=[
            pl.BlockSpec((_R1, D_FEAT), lambda i: (i, 0)),
            pl.BlockSpec((_R1, D_FEAT), lambda i: (i, 0)),
            full((D_BN, D_FEAT)),
            full((D_SHARED, D_BN)),
            full((D_BN, D_FEAT)),
            full((D_SHARED, D_BN)),
        ],
        out_specs=[
            pl.BlockSpec((_R1, D_SHARED), lambda i: (i, 0)),
            pl.BlockSpec((_R1, D_SHARED), lambda i: (i, 0)),
        ],
        out_shape=[
            jax.ShapeDtypeStruct((B, D_SHARED), jnp.float32),
            jax.ShapeDtypeStruct((B, D_SHARED), jnp.float32),
        ],
    )(vis_features, lang_features, W1_v, W2_v, W1_l, W2_l)

    out = pl.pallas_call(
        _sim_kernel,
        grid=(B // _R2,),
        in_specs=[
            pl.BlockSpec((_R2, D_SHARED), lambda i: (i, 0)),
            pl.BlockSpec((B, D_SHARED), lambda i: (0, 0)),
        ],
        out_specs=pl.BlockSpec(memory_space=pltpu.SMEM),
        out_shape=jax.ShapeDtypeStruct((2,), jnp.float32),
    )(vis_proj, lang_proj)
    return out


# speedup vs baseline: 1.2191x; 1.0332x over previous
import functools

import jax
import jax.numpy as jnp
from jax.experimental import pallas as pl
from jax.experimental.pallas import tpu as pltpu

B = 4096
D_FEAT = 1024
D_BN = 256
D_SHARED = 256

_R1 = 512
_R2 = 512

_DN = (((1,), (1,)), ((), ()))


def _proj_rows(x, W1, W2):
    h = jax.lax.dot_general(x.astype(jnp.bfloat16), W1.astype(jnp.bfloat16),
                            _DN, preferred_element_type=jnp.float32)
    h = jnp.maximum(h, 0.0)
    o = jax.lax.dot_general(h.astype(jnp.bfloat16), W2.astype(jnp.bfloat16),
                            _DN, preferred_element_type=jnp.float32)
    n = jnp.sqrt(jnp.sum(o * o, axis=-1, keepdims=True))
    return (o / jnp.maximum(n, 1e-12)).astype(jnp.bfloat16)


def _project_kernel(vis_ref, lang_ref, w1v_ref, w2v_ref, w1l_ref, w2l_ref,
                    vout_ref, lout_ref):
    vout_ref[...] = _proj_rows(vis_ref[...], w1v_ref[...], w2v_ref[...])
    lout_ref[...] = _proj_rows(lang_ref[...], w1l_ref[...], w2l_ref[...])


def _sim_kernel(vp_ref, lp_ref, out_ref):
    i = pl.program_id(0)

    @pl.when(i == 0)
    def _init():
        out_ref[0] = 0.0
        out_ref[1] = 0.0

    sim = jax.lax.dot_general(vp_ref[...], lp_ref[...], _DN,
                              preferred_element_type=jnp.float32)
    rows = i * _R2 + jax.lax.broadcasted_iota(jnp.int32, (_R2, B), 0)
    cols = jax.lax.broadcasted_iota(jnp.int32, (_R2, B), 1)
    eye = rows == cols
    pos = jnp.sum(jnp.where(eye, sim, 0.0), axis=1)
    neg = jnp.max(jnp.where(eye, -1e9, sim), axis=1)
    out_ref[0] = out_ref[0] + jnp.sum(pos) * (1.0 / B)
    out_ref[1] = out_ref[1] + jnp.sum(pos - neg) * (1.0 / B)


@jax.jit
def kernel(vis_features, lang_features, W1_v, W2_v, W1_l, W2_l):
    full = lambda shape: pl.BlockSpec(shape, lambda i: (0, 0))
    vis_proj, lang_proj = pl.pallas_call(
        _project_kernel,
        grid=(B // _R1,),
        in_specs=[
            pl.BlockSpec((_R1, D_FEAT), lambda i: (i, 0)),
            pl.BlockSpec((_R1, D_FEAT), lambda i: (i, 0)),
            full((D_BN, D_FEAT)),
            full((D_SHARED, D_BN)),
            full((D_BN, D_FEAT)),
            full((D_SHARED, D_BN)),
        ],
        out_specs=[
            pl.BlockSpec((_R1, D_SHARED), lambda i: (i, 0)),
            pl.BlockSpec((_R1, D_SHARED), lambda i: (i, 0)),
        ],
        out_shape=[
            jax.ShapeDtypeStruct((B, D_SHARED), jnp.bfloat16),
            jax.ShapeDtypeStruct((B, D_SHARED), jnp.bfloat16),
        ],
    )(vis_features, lang_features, W1_v, W2_v, W1_l, W2_l)

    out = pl.pallas_call(
        _sim_kernel,
        grid=(B // _R2,),
        in_specs=[
            pl.BlockSpec((_R2, D_SHARED), lambda i: (i, 0)),
            pl.BlockSpec((B, D_SHARED), lambda i: (0, 0)),
        ],
        out_specs=pl.BlockSpec(memory_space=pltpu.SMEM),
        out_shape=jax.ShapeDtypeStruct((2,), jnp.float32),
    )(vis_proj, lang_proj)
    return out


# speedup vs baseline: 1.2351x; 1.0132x over previous
import functools

import jax
import jax.numpy as jnp
from jax.experimental import pallas as pl
from jax.experimental.pallas import tpu as pltpu

B = 4096
D_FEAT = 1024
D_BN = 256
D_SHARED = 256

_R1 = 512
_R2 = 512

_DN = (((1,), (1,)), ((), ()))


def _proj_rows(x, W1, W2):
    h = jax.lax.dot_general(x.astype(jnp.bfloat16), W1.astype(jnp.bfloat16),
                            _DN, preferred_element_type=jnp.float32)
    h = jnp.maximum(h, 0.0)
    o = jax.lax.dot_general(h.astype(jnp.bfloat16), W2.astype(jnp.bfloat16),
                            _DN, preferred_element_type=jnp.float32)
    n = jnp.sqrt(jnp.sum(o * o, axis=-1, keepdims=True))
    return (o / jnp.maximum(n, 1e-12)).astype(jnp.bfloat16)


def _project_kernel(vis_ref, lang_ref, w1v_ref, w2v_ref, w1l_ref, w2l_ref,
                    vout_ref, lout_ref):
    vout_ref[...] = _proj_rows(vis_ref[...], w1v_ref[...], w2v_ref[...])
    lout_ref[...] = _proj_rows(lang_ref[...], w1l_ref[...], w2l_ref[...])


def _sim_kernel(vp_ref, lp_ref, lpd_ref, out_ref, scr_ref):
    i = pl.program_id(0)

    @pl.when(i == 0)
    def _init():
        out_ref[0] = 0.0
        out_ref[1] = 0.0

    vp32 = vp_ref[...].astype(jnp.float32)
    pos = jnp.sum(vp32 * lpd_ref[...].astype(jnp.float32), axis=1)

    scr_ref[...] = jax.lax.dot_general(vp_ref[...], lp_ref[...], _DN,
                                       preferred_element_type=jnp.float32)
    eye = (jax.lax.broadcasted_iota(jnp.int32, (_R2, _R2), 0)
           == jax.lax.broadcasted_iota(jnp.int32, (_R2, _R2), 1))
    band = scr_ref[:, pl.ds(i * _R2, _R2)]
    scr_ref[:, pl.ds(i * _R2, _R2)] = jnp.where(eye, -1e9, band)
    neg = jnp.max(scr_ref[...], axis=1)
    out_ref[0] = out_ref[0] + jnp.sum(pos) * (1.0 / B)
    out_ref[1] = out_ref[1] + jnp.sum(pos - neg) * (1.0 / B)


@jax.jit
def kernel(vis_features, lang_features, W1_v, W2_v, W1_l, W2_l):
    full = lambda shape: pl.BlockSpec(shape, lambda i: (0, 0))
    vis_proj, lang_proj = pl.pallas_call(
        _project_kernel,
        grid=(B // _R1,),
        in_specs=[
            pl.BlockSpec((_R1, D_FEAT), lambda i: (i, 0)),
            pl.BlockSpec((_R1, D_FEAT), lambda i: (i, 0)),
            full((D_BN, D_FEAT)),
            full((D_SHARED, D_BN)),
            full((D_BN, D_FEAT)),
            full((D_SHARED, D_BN)),
        ],
        out_specs=[
            pl.BlockSpec((_R1, D_SHARED), lambda i: (i, 0)),
            pl.BlockSpec((_R1, D_SHARED), lambda i: (i, 0)),
        ],
        out_shape=[
            jax.ShapeDtypeStruct((B, D_SHARED), jnp.bfloat16),
            jax.ShapeDtypeStruct((B, D_SHARED), jnp.bfloat16),
        ],
    )(vis_features, lang_features, W1_v, W2_v, W1_l, W2_l)

    out = pl.pallas_call(
        _sim_kernel,
        grid=(B // _R2,),
        in_specs=[
            pl.BlockSpec((_R2, D_SHARED), lambda i: (i, 0)),
            pl.BlockSpec((B, D_SHARED), lambda i: (0, 0)),
            pl.BlockSpec((_R2, D_SHARED), lambda i: (i, 0)),
        ],
        out_specs=pl.BlockSpec(memory_space=pltpu.SMEM),
        out_shape=jax.ShapeDtypeStruct((2,), jnp.float32),
        scratch_shapes=[pltpu.VMEM((_R2, B), jnp.float32)],
    )(vis_proj, lang_proj, lang_proj)
    return out
